# baseline (speedup 1.0000x reference)
import functools

import jax
import jax.numpy as jnp
from jax import lax
from jax.experimental import pallas as pl
from jax.experimental.pallas import tpu as pltpu
from jax.experimental.pallas import tpu_sc as plsc

EMBED = 64
IDXW = 128
NSLOT = 2


def _emb_body(idx_hbm, table_hbm, out_hbm, idx_v, rows_v,
              isem0, isem1, wsem0, wsem1, gsem, rows_per_w, seq):
    isems = (isem0, isem1)
    wsems = (wsem0, wsem1)
    nc = 2
    wid = lax.axis_index("s") * nc + lax.axis_index("c")
    row_base = wid * rows_per_w
    n_pairs = rows_per_w // NSLOT
    splits = [(o, min(IDXW, seq - o)) for o in range(0, seq, IDXW)]

    def idx_start(i, b):
        pltpu.async_copy(idx_hbm.at[row_base + i], idx_v.at[b], isems[b])

    def idx_wait(b):
        pltpu.make_async_copy(idx_hbm.at[0], idx_v.at[b], isems[b]).wait()

    def gathers(b):
        cs = []
        for (o, w) in splits:
            cs.append(pltpu.async_copy(
                table_hbm.at[idx_v.at[b, pl.ds(o, w)]],
                rows_v.at[b, pl.ds(o, w)],
                gsem))
        for c in cs:
            c.wait()

    def wb_start(i, b):
        pltpu.async_copy(rows_v.at[b], out_hbm.at[row_base + i], wsems[b])

    def wb_wait(b):
        pltpu.make_async_copy(rows_v.at[b], out_hbm.at[0], wsems[b]).wait()

    idx_start(0, 0)
    idx_start(1, 1)

    for b in range(NSLOT):
        idx_wait(b)
        gathers(b)
        idx_start(b + NSLOT, b)
        wb_start(b, b)

    def pair_body(p, carry):
        for b in range(NSLOT):
            i = NSLOT * p + b
            wb_wait(b)
            idx_wait(b)
            gathers(b)
            idx_start(i + NSLOT, b)
            wb_start(i, b)
        return carry

    lax.fori_loop(1, n_pairs - 1, pair_body, 0)

    for b in range(NSLOT):
        i = NSLOT * (n_pairs - 1) + b
        wb_wait(b)
        idx_wait(b)
        gathers(b)
        wb_start(i, b)
    for b in range(NSLOT):
        wb_wait(b)


def kernel(tokens_ids, table):
    batch, seq = tokens_ids.shape
    vocab, embed = table.shape
    nw = 32
    assert embed == EMBED and batch % nw == 0 and seq % 8 == 0
    rows_per_w = batch // nw
    assert rows_per_w % NSLOT == 0 and rows_per_w >= 2 * NSLOT

    grid_kernel = pl.kernel(
        functools.partial(_emb_body, rows_per_w=rows_per_w, seq=seq),
        out_type=jax.ShapeDtypeStruct((batch, seq, embed), jnp.float32),
        mesh=plsc.VectorSubcoreMesh(core_axis_name="c", subcore_axis_name="s"),
        scratch_types=[
            pltpu.VMEM((NSLOT, seq), jnp.int32),
            pltpu.VMEM((NSLOT, seq, EMBED), jnp.float32),
            pltpu.SemaphoreType.DMA,
            pltpu.SemaphoreType.DMA,
            pltpu.SemaphoreType.DMA,
            pltpu.SemaphoreType.DMA,
            pltpu.SemaphoreType.DMA,
        ],
        compiler_params=pltpu.CompilerParams(use_tc_tiling_on_sc=False),
    )
    return grid_kernel(tokens_ids, table)


# speedup vs baseline: 1.3649x; 1.3649x over previous
import functools

import jax
import jax.numpy as jnp
from jax import lax
from jax.experimental import pallas as pl
from jax.experimental.pallas import tpu as pltpu
from jax.experimental.pallas import tpu_sc as plsc

EMBED = 64
LANES = 16
BCOLS = 128
NSLOT = 2


def _emb_body(tok_hbm, table_hbm, out_hbm, idx_v, rows_v, tr_v,
              gsem0, gsem1, wsem0, wsem1, seq):
    gsems = (gsem0, gsem1)
    wsems = (wsem0, wsem1)
    nc = 2
    wid = lax.axis_index("s") * nc + lax.axis_index("c")
    col0 = wid * BCOLS
    n_pairs = seq // NSLOT

    i16 = lax.iota(jnp.int32, LANES)
    cpats = [(i16 + d) & (LANES - 1) for d in range(LANES)]

    pltpu.sync_copy(tok_hbm.at[:, pl.ds(col0, BCOLS)], idx_v)

    def g_start(s, b):
        pltpu.async_copy(table_hbm.at[idx_v.at[s]], rows_v.at[b], gsems[b])

    def g_wait(b):
        pltpu.make_async_copy(table_hbm.at[idx_v.at[0]], rows_v.at[b],
                              gsems[b]).wait()

    def wb_start(s, b):
        pltpu.async_copy(tr_v.at[b], out_hbm.at[s, :, pl.ds(col0, BCOLS)],
                         wsems[b])

    def wb_wait(b):
        pltpu.make_async_copy(tr_v.at[b], out_hbm.at[0, :, pl.ds(col0, BCOLS)],
                              wsems[b]).wait()

    def transpose(b):
        rows = rows_v.at[b]
        tr = tr_v.at[b]

        def bb_body(bb, carry):
            rvec = i16 + bb * LANES

            def e_body(k, carry2, rvec=rvec):
                ebase = k * LANES
                for d in range(LANES):
                    evec = ebase + cpats[d]
                    val = plsc.load_gather(rows, [rvec, evec])
                    plsc.store_scatter(tr, [evec, rvec], val)
                return carry2

            lax.fori_loop(0, EMBED // LANES, e_body, 0)
            return carry

        lax.fori_loop(0, BCOLS // LANES, bb_body, 0)

    g_start(0, 0)
    g_start(1, 1)

    def pair_body(p, carry):
        for b in range(NSLOT):
            s = NSLOT * p + b
            g_wait(b)
            pl.when(p > 0)(lambda b=b: wb_wait(b))
            transpose(b)
            pl.when(p < n_pairs - 1)(lambda s=s, b=b: g_start(s + NSLOT, b))
            wb_start(s, b)
        return carry

    lax.fori_loop(0, n_pairs, pair_body, 0)

    for b in range(NSLOT):
        wb_wait(b)


def kernel(tokens_ids, table):
    batch, seq = tokens_ids.shape
    vocab, embed = table.shape
    nw = 32
    assert embed == EMBED and batch == nw * BCOLS
    assert seq % NSLOT == 0 and seq >= 2 * NSLOT

    tok_t = tokens_ids.T
    table_p = jnp.pad(table, ((0, 0), (0, 2 * EMBED - embed)))

    grid_kernel = pl.kernel(
        functools.partial(_emb_body, seq=seq),
        out_type=jax.ShapeDtypeStruct((seq, embed, batch), jnp.float32),
        mesh=plsc.VectorSubcoreMesh(core_axis_name="c", subcore_axis_name="s"),
        scratch_types=[
            pltpu.VMEM((seq, BCOLS), jnp.int32),
            pltpu.VMEM((NSLOT, BCOLS, 2 * EMBED), jnp.float32),
            pltpu.VMEM((NSLOT, EMBED, BCOLS), jnp.float32),
            pltpu.SemaphoreType.DMA,
            pltpu.SemaphoreType.DMA,
            pltpu.SemaphoreType.DMA,
            pltpu.SemaphoreType.DMA,
        ],
        compiler_params=pltpu.CompilerParams(
            use_tc_tiling_on_sc=True, needs_layout_passes=False),
    )
    out_t = grid_kernel(tok_t, table_p)
    return jnp.transpose(out_t, (2, 0, 1))
